# baseline (speedup 1.0000x reference)
import functools

import jax
import jax.numpy as jnp
from jax import lax
from jax.experimental import pallas as pl
from jax.experimental.pallas import tpu as pltpu
from jax.experimental.pallas import tpu_sc as plsc

_N = 10000
_E = 320000
_G = 8
_NP = 10240
_ND = 10256
_E2 = _E + _N
_K = 32
_B = 256
_T_SPLIT = 11264
_T_FULL = 22528
_EP = _T_SPLIT * 32
_ROWS = _NP // 16


def _make_edge_kernel(split_edges):
    T = _T_SPLIT if split_edges else _T_FULL
    mesh = plsc.VectorSubcoreMesh(core_axis_name="c", subcore_axis_name="s")

    @functools.partial(
        pl.kernel,
        mesh=mesh,
        compiler_params=pltpu.CompilerParams(needs_layout_passes=False),
        out_type=[
            jax.ShapeDtypeStruct((2 * _NP, 128), jnp.float32),
            jax.ShapeDtypeStruct((32, _ND), jnp.float32),
        ],
        scratch_types=[
            pltpu.VMEM((_NP,), jnp.float32),
            pltpu.VMEM((_NP,), jnp.float32),
            pltpu.VMEM((_B,), jnp.int32),
            pltpu.VMEM((_B,), jnp.int32),
            pltpu.VMEM((16,), jnp.float32),
            pltpu.VMEM((_K,), jnp.int32),
            pltpu.VMEM((_K,), jnp.int32),
            pltpu.VMEM((_K,), jnp.int32),
            pltpu.VMEM((_K,), jnp.int32),
            pltpu.VMEM((_K,), jnp.int32),
            pltpu.VMEM((_K,), jnp.int32),
            pltpu.VMEM((_K,), jnp.float32),
            pltpu.VMEM((_K,), jnp.float32),
            pltpu.VMEM((_ND,), jnp.float32),
            pltpu.VMEM((_K, 128), jnp.float32),
            pltpu.VMEM((_K, 128), jnp.float32),
            pltpu.VMEM((_K, 128), jnp.float32),
            pltpu.VMEM((_K, 128), jnp.float32),
            pltpu.VMEM_SHARED((_NP, 128), jnp.float32),
            pltpu.SemaphoreType.DMA,
            pltpu.SemaphoreType.DMA,
            pltpu.SemaphoreType.DMA,
            pltpu.SemaphoreType.DMA,
        ],
    )
    def edge_kernel(src_h, dst_h, asrc_h, adst_h, cmax_h, xpa_h, xpb_h,
                    zero_h, zero1_h, out_h, outd_h, asrc_v, adst_v, src_v,
                    dst_v, cmax_v, gidx0_v, gidx1_v, sidx0_v, sidx1_v,
                    ssidx0_v, ssidx1_v, wb0_v, wb1_v, den_v, rows0_v,
                    rows1_v, scaled0_v, scaled1_v, acc_s,
                    sem0, sem1, ssem0, ssem1):
        c = lax.axis_index("c")
        s = lax.axis_index("s")
        r0 = s * _ROWS
        pltpu.sync_copy(zero_h.at[pl.ds(r0, _ROWS)], acc_s.at[pl.ds(r0, _ROWS)])
        if split_edges:
            base = (c * 16 + s) * T
        else:
            base = s * T
        pltpu.sync_copy(asrc_h, asrc_v)
        pltpu.sync_copy(adst_h, adst_v)
        pltpu.sync_copy(cmax_h, cmax_v)
        pltpu.sync_copy(zero1_h, den_v)
        plsc.subcore_barrier()
        cmax = cmax_v[...][0]
        lane = lax.iota(jnp.int32, 16)
        nchunks = _B // _K

        def prepare(boff, off, gidx_v, sidx_v, wb_v):
            for sub in range(_K // 16):
                o2 = off + sub * 16
                sv = src_v[pl.ds(o2, 16)]
                dv = dst_v[pl.ds(o2, 16)]
                av = (plsc.load_gather(asrc_v, [sv])
                      + plsc.load_gather(adst_v, [dv]))
                av = jnp.where(av > 0.0, av, 0.2 * av)
                w = jnp.exp(av - cmax)
                eid = boff + o2 + lane
                w = jnp.where(eid < _E2, w, 0.0)
                gidx_v[pl.ds(sub * 16, 16)] = sv
                sidx_v[pl.ds(sub * 16, 16)] = dv
                wb_v[pl.ds(sub * 16, 16)] = w
                plsc.addupdate_scatter(den_v, [dv], w)

        def start(gidx_v, rows_v, sem):
            @pl.when(c == 0)
            def _():
                pltpu.async_copy(xpa_h.at[gidx_v], rows_v, sem)

            @pl.when(c == 1)
            def _():
                pltpu.async_copy(xpb_h.at[gidx_v], rows_v, sem)

        def consume(not_first, gidx_v, sidx_v, wb_v, rows_v, sem,
                    scaled_v, ssidx_v, ssem):
            pltpu.make_async_copy(xpa_h.at[gidx_v], rows_v, sem).wait()

            @pl.when(not_first)
            def _():
                pltpu.make_async_copy(scaled_v, acc_s.at[ssidx_v],
                                      ssem).wait()

            for g in range(_K // 16):
                ssidx_v[pl.ds(g * 16, 16)] = sidx_v[pl.ds(g * 16, 16)]
                wv = wb_v[pl.ds(g * 16, 16)]
                for l in range(16):
                    e = g * 16 + l
                    we = wv[l]
                    for j in range(8):
                        scaled_v[e, pl.ds(j * 16, 16)] = (
                            rows_v[e, pl.ds(j * 16, 16)] * we)
            pltpu.async_copy(scaled_v, acc_s.at[ssidx_v], ssem, add=True)

        def block(bi, carry):
            boff = base + bi * _B
            pltpu.sync_copy(src_h.at[pl.ds(boff, _B)], src_v)
            pltpu.sync_copy(dst_h.at[pl.ds(boff, _B)], dst_v)
            prepare(boff, 0, gidx0_v, sidx0_v, wb0_v)
            start(gidx0_v, rows0_v, sem0)

            def pair(p, carry2):
                not_first = jnp.logical_or(bi > 0, p > 0)
                off0 = (2 * p) * _K
                prepare(boff, off0 + _K, gidx1_v, sidx1_v, wb1_v)
                start(gidx1_v, rows1_v, sem1)
                consume(not_first, gidx0_v, sidx0_v, wb0_v, rows0_v, sem0,
                        scaled0_v, ssidx0_v, ssem0)

                @pl.when(2 * p + 2 < nchunks)
                def _():
                    prepare(boff, off0 + 2 * _K, gidx0_v, sidx0_v, wb0_v)
                    start(gidx0_v, rows0_v, sem0)

                consume(not_first, gidx1_v, sidx1_v, wb1_v, rows1_v, sem1,
                        scaled1_v, ssidx1_v, ssem1)
                return carry2

            lax.fori_loop(0, nchunks // 2, pair, 0)
            return carry

        lax.fori_loop(0, T // _B, block, 0)
        pltpu.make_async_copy(scaled0_v, acc_s.at[ssidx0_v], ssem0).wait()
        pltpu.make_async_copy(scaled1_v, acc_s.at[ssidx1_v], ssem1).wait()
        pltpu.sync_copy(den_v, outd_h.at[c * 16 + s])
        plsc.subcore_barrier()
        pltpu.sync_copy(acc_s.at[pl.ds(r0, _ROWS)],
                        out_h.at[pl.ds(c * _NP + r0, _ROWS)])

    return edge_kernel


_edge_split = _make_edge_kernel(True)
_edge_full = _make_edge_kernel(False)


def _denom_col(d2, nrows):
    ones = jnp.ones((nrows, 1), jnp.float32)
    col = lax.dot_general(d2[:nrows, :], ones, (((0,), (0,)), ((), ())),
                          preferred_element_type=jnp.float32)
    return col[:_NP, :]


def _alpha_outs(xp, as_ref, ad_ref, a1_ref, a2_ref, c_ref):
    a1 = jnp.sum(xp * as_ref[...], axis=1, keepdims=True)
    a2 = jnp.sum(xp * ad_ref[...], axis=1, keepdims=True)
    a1_ref[...] = a1
    a2_ref[...] = a2
    c_ref[...] = (jnp.max(a1) + jnp.max(a2)).reshape(1, 1)


def _tc_first_body(x_ref, w_ref, as_ref, ad_ref,
                   xp_ref, a1_ref, a2_ref, c_ref):
    xp = jnp.dot(x_ref[...], w_ref[...], preferred_element_type=jnp.float32)
    xp_ref[...] = xp
    _alpha_outs(xp, as_ref, ad_ref, a1_ref, a2_ref, c_ref)


def _make_tc_mid_body(dout):
    def body(sc_ref, d2_ref, b_ref, w_ref, as_ref, ad_ref, *outs):
        n = sc_ref[0] + sc_ref[1]
        dnm = _denom_col(d2_ref[...], 32)
        h = jnp.maximum(n / (dnm + 1e-16) + b_ref[...], 0.0)
        xp = jnp.dot(h, w_ref[...], preferred_element_type=jnp.float32)
        if dout == 128:
            xp_ref, a1_ref, a2_ref, c_ref = outs
            xp_ref[...] = xp
        else:
            xlo_ref, xhi_ref, a1_ref, a2_ref, c_ref = outs
            xlo_ref[...] = xp[:, :128]
            xhi_ref[...] = xp[:, 128:]
        _alpha_outs(xp, as_ref, ad_ref, a1_ref, a2_ref, c_ref)
    return body


def _tc_final_body(sc_ref, d2_ref, b_ref, batch_ref, out_ref):
    n = jnp.concatenate([sc_ref[0], sc_ref[1]], axis=1)
    dnm = _denom_col(d2_ref[...], 16)
    h = n / (dnm + 1e-16) + b_ref[...]
    gi = lax.broadcasted_iota(jnp.int32, (_G, _NP), 0)
    m = (batch_ref[...] == gi).astype(jnp.float32)
    ssum = jnp.dot(m, h, preferred_element_type=jnp.float32)
    cnt = jnp.sum(m, axis=1, keepdims=True)
    mean = ssum / jnp.maximum(cnt, 1.0)
    nrm = jnp.sqrt(jnp.sum(mean * mean, axis=1, keepdims=True))
    out_ref[...] = mean / jnp.maximum(nrm, 1e-12)


def _f(shape):
    return jax.ShapeDtypeStruct(shape, jnp.float32)


def _cvec(c):
    return jnp.broadcast_to(jnp.squeeze(c), (16,)).astype(jnp.float32)


def kernel(x, edge_index, batch, W1, a_src1, a_dst1, b1,
           W2, a_src2, a_dst2, b2, W3, a_src3, a_dst3, b3):
    f32 = jnp.float32
    xpad = jnp.pad(x.astype(f32), ((0, _NP - _N), (0, 0)))
    node_ids = jnp.arange(_N, dtype=jnp.int32)
    pad_e = (jnp.arange(_EP - _E2, dtype=jnp.int32) % 224) + _N
    src = jnp.concatenate([edge_index[0].astype(jnp.int32), node_ids, pad_e])
    dst = jnp.concatenate([edge_index[1].astype(jnp.int32), node_ids, pad_e])
    batch_p = jnp.pad(batch.astype(jnp.int32), (0, _NP - _N),
                      constant_values=_G).reshape(1, _NP)
    zeros = jnp.zeros((_NP, 128), f32)
    zeros1 = jnp.zeros((_ND,), f32)

    xp1, a11, a21, c1 = pl.pallas_call(
        _tc_first_body,
        out_shape=[_f((_NP, 128)), _f((_NP, 1)), _f((_NP, 1)), _f((1, 1))],
    )(xpad, W1, a_src1.reshape(1, -1), a_dst1.reshape(1, -1))
    sc1, d1 = _edge_split(src, dst, a11.reshape(_NP), a21.reshape(_NP),
                          _cvec(c1), xp1, xp1, zeros, zeros1)

    xp2, a12, a22, c2 = pl.pallas_call(
        _make_tc_mid_body(128),
        out_shape=[_f((_NP, 128)), _f((_NP, 1)), _f((_NP, 1)), _f((1, 1))],
    )(sc1.reshape(2, _NP, 128), d1, b1.reshape(1, -1), W2,
      a_src2.reshape(1, -1), a_dst2.reshape(1, -1))
    sc2, d2 = _edge_split(src, dst, a12.reshape(_NP), a22.reshape(_NP),
                          _cvec(c2), xp2, xp2, zeros, zeros1)

    xlo, xhi, a13, a23, c3 = pl.pallas_call(
        _make_tc_mid_body(256),
        out_shape=[_f((_NP, 128)), _f((_NP, 128)), _f((_NP, 1)), _f((_NP, 1)),
                   _f((1, 1))],
    )(sc2.reshape(2, _NP, 128), d2, b2.reshape(1, -1), W3,
      a_src3.reshape(1, -1), a_dst3.reshape(1, -1))
    sc3, d3 = _edge_full(src, dst, a13.reshape(_NP), a23.reshape(_NP),
                         _cvec(c3), xlo, xhi, zeros, zeros1)

    out = pl.pallas_call(
        _tc_final_body,
        out_shape=_f((_G, 256)),
    )(sc3.reshape(2, _NP, 128), d3, b3.reshape(1, -1), batch_p)
    return out


# speedup vs baseline: 1.0662x; 1.0662x over previous
import functools

import jax
import jax.numpy as jnp
from jax import lax
from jax.experimental import pallas as pl
from jax.experimental.pallas import tpu as pltpu
from jax.experimental.pallas import tpu_sc as plsc

_N = 10000
_E = 320000
_G = 8
_NP = 10240
_ND = 10256
_E2 = _E + _N
_K = 32
_B = 256
_T_SPLIT = 10496
_T_FULL = 20736
_EP = _T_SPLIT * 32
_ROWS = _NP // 16


def _make_edge_kernel(split_edges):
    T = _T_SPLIT if split_edges else _T_FULL
    mesh = plsc.VectorSubcoreMesh(core_axis_name="c", subcore_axis_name="s")

    @functools.partial(
        pl.kernel,
        mesh=mesh,
        compiler_params=pltpu.CompilerParams(needs_layout_passes=False),
        out_type=[
            jax.ShapeDtypeStruct((2 * _NP, 128), jnp.float32),
            jax.ShapeDtypeStruct((32, _ND), jnp.float32),
        ],
        scratch_types=[
            pltpu.VMEM((_NP,), jnp.float32),
            pltpu.VMEM((_NP,), jnp.float32),
            pltpu.VMEM((_B,), jnp.int32),
            pltpu.VMEM((_B,), jnp.int32),
            pltpu.VMEM((16,), jnp.float32),
            pltpu.VMEM((_K,), jnp.int32),
            pltpu.VMEM((_K,), jnp.int32),
            pltpu.VMEM((_K,), jnp.int32),
            pltpu.VMEM((_K,), jnp.int32),
            pltpu.VMEM((_K,), jnp.int32),
            pltpu.VMEM((_K,), jnp.int32),
            pltpu.VMEM((_K,), jnp.float32),
            pltpu.VMEM((_K,), jnp.float32),
            pltpu.VMEM((_ND,), jnp.float32),
            pltpu.VMEM((_K, 128), jnp.float32),
            pltpu.VMEM((_K, 128), jnp.float32),
            pltpu.VMEM((_K, 128), jnp.float32),
            pltpu.VMEM((_K, 128), jnp.float32),
            pltpu.VMEM_SHARED((_NP, 128), jnp.float32),
            pltpu.SemaphoreType.DMA,
            pltpu.SemaphoreType.DMA,
            pltpu.SemaphoreType.DMA,
            pltpu.SemaphoreType.DMA,
        ],
    )
    def edge_kernel(src_h, dst_h, asrc_h, adst_h, cmax_h, xpa_h, xpb_h,
                    zero_h, zero1_h, out_h, outd_h, asrc_v, adst_v, src_v,
                    dst_v, cmax_v, gidx0_v, gidx1_v, sidx0_v, sidx1_v,
                    ssidx0_v, ssidx1_v, wb0_v, wb1_v, den_v, rows0_v,
                    rows1_v, scaled0_v, scaled1_v, acc_s,
                    sem0, sem1, ssem0, ssem1):
        c = lax.axis_index("c")
        s = lax.axis_index("s")
        r0 = s * _ROWS
        pltpu.sync_copy(zero_h.at[pl.ds(r0, _ROWS)], acc_s.at[pl.ds(r0, _ROWS)])
        if split_edges:
            base = (c * 16 + s) * T
        else:
            base = s * T
        pltpu.sync_copy(asrc_h, asrc_v)
        pltpu.sync_copy(adst_h, adst_v)
        pltpu.sync_copy(cmax_h, cmax_v)
        pltpu.sync_copy(zero1_h, den_v)
        plsc.subcore_barrier()
        cmax = cmax_v[...][0]
        lane = lax.iota(jnp.int32, 16)
        nchunks = _B // _K

        def prepare(boff, off, gidx_v, sidx_v, wb_v):
            for sub in range(_K // 16):
                o2 = off + sub * 16
                sv = src_v[pl.ds(o2, 16)]
                dv = dst_v[pl.ds(o2, 16)]
                av = (plsc.load_gather(asrc_v, [sv])
                      + plsc.load_gather(adst_v, [dv]))
                av = jnp.where(av > 0.0, av, 0.2 * av)
                w = jnp.exp(av - cmax)
                eid = boff + o2 + lane
                w = jnp.where(eid < _E2, w, 0.0)
                gidx_v[pl.ds(sub * 16, 16)] = sv
                sidx_v[pl.ds(sub * 16, 16)] = dv
                wb_v[pl.ds(sub * 16, 16)] = w
                plsc.addupdate_scatter(den_v, [dv], w)

        def start(gidx_v, rows_v, sem):
            @pl.when(c == 0)
            def _():
                pltpu.async_copy(xpa_h.at[gidx_v], rows_v, sem)

            @pl.when(c == 1)
            def _():
                pltpu.async_copy(xpb_h.at[gidx_v], rows_v, sem)

        def consume(not_first, gidx_v, sidx_v, wb_v, rows_v, sem,
                    scaled_v, ssidx_v, ssem):
            pltpu.make_async_copy(xpa_h.at[gidx_v], rows_v, sem).wait()

            @pl.when(not_first)
            def _():
                pltpu.make_async_copy(scaled_v, acc_s.at[ssidx_v],
                                      ssem).wait()

            for g in range(_K // 16):
                ssidx_v[pl.ds(g * 16, 16)] = sidx_v[pl.ds(g * 16, 16)]
                wv = wb_v[pl.ds(g * 16, 16)]
                for l in range(16):
                    e = g * 16 + l
                    we = wv[l]
                    for j in range(8):
                        scaled_v[e, pl.ds(j * 16, 16)] = (
                            rows_v[e, pl.ds(j * 16, 16)] * we)
            pltpu.async_copy(scaled_v, acc_s.at[ssidx_v], ssem, add=True)

        def block(bi, carry):
            boff = base + bi * _B
            pltpu.sync_copy(src_h.at[pl.ds(boff, _B)], src_v)
            pltpu.sync_copy(dst_h.at[pl.ds(boff, _B)], dst_v)
            prepare(boff, 0, gidx0_v, sidx0_v, wb0_v)
            start(gidx0_v, rows0_v, sem0)

            def pair(p, carry2):
                not_first = jnp.logical_or(bi > 0, p > 0)
                off0 = (2 * p) * _K
                prepare(boff, off0 + _K, gidx1_v, sidx1_v, wb1_v)
                start(gidx1_v, rows1_v, sem1)
                consume(not_first, gidx0_v, sidx0_v, wb0_v, rows0_v, sem0,
                        scaled0_v, ssidx0_v, ssem0)

                @pl.when(2 * p + 2 < nchunks)
                def _():
                    prepare(boff, off0 + 2 * _K, gidx0_v, sidx0_v, wb0_v)
                    start(gidx0_v, rows0_v, sem0)

                consume(not_first, gidx1_v, sidx1_v, wb1_v, rows1_v, sem1,
                        scaled1_v, ssidx1_v, ssem1)
                return carry2

            lax.fori_loop(0, nchunks // 2, pair, 0)
            return carry

        lax.fori_loop(0, T // _B, block, 0)
        pltpu.make_async_copy(scaled0_v, acc_s.at[ssidx0_v], ssem0).wait()
        pltpu.make_async_copy(scaled1_v, acc_s.at[ssidx1_v], ssem1).wait()
        pltpu.sync_copy(den_v, outd_h.at[c * 16 + s])
        plsc.subcore_barrier()
        pltpu.sync_copy(acc_s.at[pl.ds(r0, _ROWS)],
                        out_h.at[pl.ds(c * _NP + r0, _ROWS)])

    return edge_kernel


_edge_split = _make_edge_kernel(True)
_edge_full = _make_edge_kernel(False)


def _denom_col(d2, nrows):
    ones = jnp.ones((nrows, 1), jnp.float32)
    col = lax.dot_general(d2[:nrows, :], ones, (((0,), (0,)), ((), ())),
                          preferred_element_type=jnp.float32)
    return col[:_NP, :]


def _alpha_outs(xp, as_ref, ad_ref, a1_ref, a2_ref, c_ref):
    a1 = jnp.sum(xp * as_ref[...], axis=1, keepdims=True)
    a2 = jnp.sum(xp * ad_ref[...], axis=1, keepdims=True)
    a1_ref[...] = a1
    a2_ref[...] = a2
    c_ref[...] = (jnp.max(a1) + jnp.max(a2)).reshape(1, 1)


def _tc_first_body(x_ref, w_ref, as_ref, ad_ref,
                   xp_ref, a1_ref, a2_ref, c_ref):
    xp = jnp.dot(x_ref[...], w_ref[...], preferred_element_type=jnp.float32)
    xp_ref[...] = xp
    _alpha_outs(xp, as_ref, ad_ref, a1_ref, a2_ref, c_ref)


def _make_tc_mid_body(dout):
    def body(sc_ref, d2_ref, b_ref, w_ref, as_ref, ad_ref, *outs):
        n = sc_ref[0] + sc_ref[1]
        dnm = _denom_col(d2_ref[...], 32)
        h = jnp.maximum(n / (dnm + 1e-16) + b_ref[...], 0.0)
        xp = jnp.dot(h, w_ref[...], preferred_element_type=jnp.float32)
        if dout == 128:
            xp_ref, a1_ref, a2_ref, c_ref = outs
            xp_ref[...] = xp
        else:
            xlo_ref, xhi_ref, a1_ref, a2_ref, c_ref = outs
            xlo_ref[...] = xp[:, :128]
            xhi_ref[...] = xp[:, 128:]
        _alpha_outs(xp, as_ref, ad_ref, a1_ref, a2_ref, c_ref)
    return body


def _tc_final_body(sc_ref, d2_ref, b_ref, batch_ref, out_ref):
    n = jnp.concatenate([sc_ref[0], sc_ref[1]], axis=1)
    dnm = _denom_col(d2_ref[...], 16)
    h = n / (dnm + 1e-16) + b_ref[...]
    gi = lax.broadcasted_iota(jnp.int32, (_G, _NP), 0)
    m = (batch_ref[...] == gi).astype(jnp.float32)
    ssum = jnp.dot(m, h, preferred_element_type=jnp.float32)
    cnt = jnp.sum(m, axis=1, keepdims=True)
    mean = ssum / jnp.maximum(cnt, 1.0)
    nrm = jnp.sqrt(jnp.sum(mean * mean, axis=1, keepdims=True))
    out_ref[...] = mean / jnp.maximum(nrm, 1e-12)


def _f(shape):
    return jax.ShapeDtypeStruct(shape, jnp.float32)


def _cvec(c):
    return jnp.broadcast_to(jnp.squeeze(c), (16,)).astype(jnp.float32)


def kernel(x, edge_index, batch, W1, a_src1, a_dst1, b1,
           W2, a_src2, a_dst2, b2, W3, a_src3, a_dst3, b3):
    f32 = jnp.float32
    xpad = jnp.pad(x.astype(f32), ((0, _NP - _N), (0, 0)))
    node_ids = jnp.arange(_N, dtype=jnp.int32)
    pad_e = (jnp.arange(_EP - _E2, dtype=jnp.int32) % 224) + _N
    src = jnp.concatenate([edge_index[0].astype(jnp.int32), node_ids, pad_e])
    dst = jnp.concatenate([edge_index[1].astype(jnp.int32), node_ids, pad_e])
    batch_p = jnp.pad(batch.astype(jnp.int32), (0, _NP - _N),
                      constant_values=_G).reshape(1, _NP)
    zeros = jnp.zeros((_NP, 128), f32)
    zeros1 = jnp.zeros((_ND,), f32)

    xp1, a11, a21, c1 = pl.pallas_call(
        _tc_first_body,
        out_shape=[_f((_NP, 128)), _f((_NP, 1)), _f((_NP, 1)), _f((1, 1))],
    )(xpad, W1, a_src1.reshape(1, -1), a_dst1.reshape(1, -1))
    sc1, d1 = _edge_split(src, dst, a11.reshape(_NP), a21.reshape(_NP),
                          _cvec(c1), xp1, xp1, zeros, zeros1)

    xp2, a12, a22, c2 = pl.pallas_call(
        _make_tc_mid_body(128),
        out_shape=[_f((_NP, 128)), _f((_NP, 1)), _f((_NP, 1)), _f((1, 1))],
    )(sc1.reshape(2, _NP, 128), d1, b1.reshape(1, -1), W2,
      a_src2.reshape(1, -1), a_dst2.reshape(1, -1))
    sc2, d2 = _edge_split(src, dst, a12.reshape(_NP), a22.reshape(_NP),
                          _cvec(c2), xp2, xp2, zeros, zeros1)

    xlo, xhi, a13, a23, c3 = pl.pallas_call(
        _make_tc_mid_body(256),
        out_shape=[_f((_NP, 128)), _f((_NP, 128)), _f((_NP, 1)), _f((_NP, 1)),
                   _f((1, 1))],
    )(sc2.reshape(2, _NP, 128), d2, b2.reshape(1, -1), W3,
      a_src3.reshape(1, -1), a_dst3.reshape(1, -1))
    sc3, d3 = _edge_full(src, dst, a13.reshape(_NP), a23.reshape(_NP),
                         _cvec(c3), xlo, xhi, zeros, zeros1)

    out = pl.pallas_call(
        _tc_final_body,
        out_shape=_f((_G, 256)),
    )(sc3.reshape(2, _NP, 128), d3, b3.reshape(1, -1), batch_p)
    return out
